# Initial kernel scaffold; baseline (speedup 1.0000x reference)
import jax
import jax.numpy as jnp
from jax.experimental import pallas as pl


def kernel(feat, scores, proposals, W1, b1, W2, b2, Wc, bc, Wr, br):
    raise NotImplementedError("write your pallas kernel here")


# speedup vs baseline: 1.2215x; 1.2215x over previous
import jax
import jax.numpy as jnp
from jax.experimental import pallas as pl
from jax.experimental.pallas import tpu as pltpu

_BK = 896
_D_IN = 12544
_NK = _D_IN // _BK
_TOPK = 1000


def _fc1_body(A_ref, W1_ref, b1_ref, x1_ref):
    k = pl.program_id(0)

    @pl.when(k == 0)
    def _init():
        x1_ref[...] = jnp.zeros_like(x1_ref)

    x1_ref[...] += jnp.dot(A_ref[...], W1_ref[...],
                           preferred_element_type=jnp.float32)

    @pl.when(k == _NK - 1)
    def _finish():
        x1_ref[...] = jnp.maximum(x1_ref[...] + b1_ref[...], 0.0)


def _epilogue_body(x1_ref, W2_ref, b2_ref, Wh_ref, bh_ref, sc_ref, pr_ref,
                   out_ref):
    N = x1_ref.shape[0]
    x2 = jnp.maximum(
        jnp.dot(x1_ref[...], W2_ref[...], preferred_element_type=jnp.float32)
        + b2_ref[...], 0.0)
    head = jax.lax.dot_general(
        x2, Wh_ref[...], (((1,), (0,)), ((), ())),
        precision=jax.lax.Precision.HIGHEST,
        preferred_element_type=jnp.float32) + bh_ref[...]

    pt = jnp.transpose(pr_ref[0])
    x0 = pt[0:1, :]
    y0 = pt[1:2, :]
    x1v = pt[2:3, :]
    y1v = pt[3:4, :]
    x2v = pt[4:5, :]
    y2v = pt[5:6, :]
    cx = (x0 + x1v + x2v + pt[6:7, :]) * 0.25
    cy = (y0 + y1v + y2v + pt[7:8, :]) * 0.25
    e1x = x1v - x0
    e1y = y1v - y0
    e2x = x2v - x1v
    e2y = y2v - y1v
    wv = jnp.sqrt(e1x * e1x + e1y * e1y + 1e-8)
    hv = jnp.sqrt(e2x * e2x + e2y * e2y + 1e-8)
    th = jnp.arctan2(e1y, e1x)
    enc = jnp.concatenate([cx, cy, wv, hv, th], axis=0)

    s_row = sc_ref[0]
    s_col = jnp.transpose(s_row)
    jj = jax.lax.broadcasted_iota(jnp.int32, (N, N), 0)
    ii = jax.lax.broadcasted_iota(jnp.int32, (N, N), 1)
    beats = (s_col > s_row) | ((s_col == s_row) & (jj < ii))
    rank = jnp.sum(beats.astype(jnp.int32), axis=0, keepdims=True)
    P = (rank == jj).astype(jnp.float32)
    out16 = jax.lax.dot_general(
        P, head, (((1,), (0,)), ((), ())),
        precision=jax.lax.Precision.HIGHEST,
        preferred_element_type=jnp.float32)
    oenc = jax.lax.dot_general(
        P, enc, (((1,), (1,)), ((), ())),
        precision=jax.lax.Precision.HIGHEST,
        preferred_element_type=jnp.float32)
    pad = jnp.zeros((N, 11), jnp.float32)
    out_ref[0] = jnp.concatenate([out16, oenc, pad], axis=1)


def kernel(feat, scores, proposals, W1, b1, W2, b2, Wc, bc, Wr, br):
    B, N = scores.shape
    d_in = W1.shape[0]
    d_hid = W1.shape[1]
    A = feat.reshape(B * N, d_in)
    props = proposals.reshape(B, N, 8)
    Wh = jnp.concatenate([Wc, Wr], axis=1)
    bh = jnp.concatenate([bc, br])[None, :]
    k = min(_TOPK, N)

    x1 = pl.pallas_call(
        _fc1_body,
        grid=(_NK,),
        in_specs=[
            pl.BlockSpec((B * N, _BK), lambda i: (0, i)),
            pl.BlockSpec((_BK, d_hid), lambda i: (i, 0)),
            pl.BlockSpec((1, d_hid), lambda i: (0, 0)),
        ],
        out_specs=pl.BlockSpec((B * N, d_hid), lambda i: (0, 0)),
        out_shape=jax.ShapeDtypeStruct((B * N, d_hid), jnp.float32),
        compiler_params=pltpu.CompilerParams(
            dimension_semantics=("arbitrary",)),
    )(A, W1, b1[None, :])

    out = pl.pallas_call(
        _epilogue_body,
        grid=(B,),
        in_specs=[
            pl.BlockSpec((N, d_hid), lambda b: (b, 0)),
            pl.BlockSpec((d_hid, d_hid), lambda b: (0, 0)),
            pl.BlockSpec((1, d_hid), lambda b: (0, 0)),
            pl.BlockSpec((d_hid, 16), lambda b: (0, 0)),
            pl.BlockSpec((1, 16), lambda b: (0, 0)),
            pl.BlockSpec((1, 1, N), lambda b: (b, 0, 0)),
            pl.BlockSpec((1, N, 8), lambda b: (b, 0, 0)),
        ],
        out_specs=pl.BlockSpec((1, N, 32), lambda b: (b, 0, 0)),
        out_shape=jax.ShapeDtypeStruct((B, N, 32), jnp.float32),
        compiler_params=pltpu.CompilerParams(
            dimension_semantics=("arbitrary",)),
    )(x1, W2, b2[None, :], Wh, bh, scores[:, None, :], props)

    return out[:, :k, :21]
